# baseline (speedup 1.0000x reference)
import numpy as np
import jax
import jax.numpy as jnp
from jax import lax
from jax.experimental import pallas as pl
from jax.experimental.pallas import tpu as pltpu

BT = 256
LN = 128


def _net_kernel(x_ref, band_ref, b1_ref, w2_ref, b2_ref,
                wf1_ref, bf1_ref, wf2_ref, bf2_ref, out_ref,
                h1_ref, h2_ref):
    hp = lax.Precision.DEFAULT
    f32 = jnp.float32
    X = x_ref[...].reshape(28 * BT, 28).astype(jnp.bfloat16)

    for r in range(12):
        acc = jnp.zeros((2 * BT, 256), f32)
        for ki in range(5):
            acc = acc + jnp.dot(X[(2 * r + ki) * BT:(2 * r + ki + 2) * BT],
                                band_ref[ki],
                                precision=hp, preferred_element_type=f32)
        v = acc.reshape(2, BT, 256)
        m = jnp.maximum(v[0], v[1])
        m = jnp.maximum(m[:, :LN], m[:, LN:])
        h1_ref[r * BT:(r + 1) * BT] = jnp.maximum(m + b1_ref[...], 0.0)

    for rr in range(4):
        acc = jnp.zeros((2 * BT, 256), f32)
        for ki in range(5):
            acc = acc + jnp.dot(
                h1_ref[(2 * rr + ki) * BT:(2 * rr + ki + 2) * BT], w2_ref[ki],
                precision=hp, preferred_element_type=f32)
        v = acc.reshape(2, BT, 256)
        m = jnp.maximum(v[0], v[1])
        m = jnp.maximum(m[:, :LN], m[:, LN:])
        h2_ref[rr * BT:(rr + 1) * BT] = jnp.maximum(m + b2_ref[...], 0.0)

    accf = jnp.zeros((BT, LN), f32)
    for ii in range(4):
        accf = accf + jnp.dot(h2_ref[ii * BT:(ii + 1) * BT], wf1_ref[ii],
                              precision=hp, preferred_element_type=f32)
    f1 = jnp.maximum(accf + bf1_ref[...], 0.0)

    out_ref[...] = jnp.dot(f1, wf2_ref[...], precision=hp,
                           preferred_element_type=f32) + bf2_ref[...]


def _log_softmax_dim0_kernel(x_ref, o_ref):
    x = x_ref[...]
    m = jnp.max(x, axis=0, keepdims=True)
    s = x - m
    lse = jnp.log(jnp.sum(jnp.exp(s), axis=0, keepdims=True))
    o_ref[...] = (s - lse)[:, :10]


def _onehot(w_size, m_size, parity):
    kj = np.arange(5)[:, None, None]
    w = np.arange(w_size)[None, :, None]
    m = np.arange(m_size)[None, None, :]
    return (w == 2 * m + parity + kj).astype(np.float32)


def _build_band1(w1):
    w1r = w1[:, 0]
    halves = []
    for parity in (0, 1):
        e = _onehot(28, 12, parity)
        h = jnp.einsum('cij,jwm->iwcm', w1r, e).reshape(5, 28, 120)
        halves.append(jnp.pad(h, ((0, 0), (0, 0), (0, LN - 120))))
    return jnp.concatenate(halves, axis=2)


def _build_band2(w2):
    halves = []
    for parity in (0, 1):
        e = _onehot(12, 4, parity)
        h = jnp.einsum('ocij,jwm->icwom', w2, e).reshape(5, 120, 80)
        halves.append(jnp.pad(h, ((0, 0), (0, 8), (0, LN - 80))))
    return jnp.concatenate(halves, axis=2)


def kernel(x, conv1_w, conv1_b, conv2_w, conv2_b, fc1_w, fc1_b, fc2_w, fc2_b):
    B = x.shape[0]
    n_tiles = (B + BT - 1) // BT
    b_pad = n_tiles * BT

    xt = jnp.transpose(x.reshape(B, 28, 28), (1, 0, 2))
    if b_pad != B:
        xt = jnp.pad(xt, ((0, 0), (0, b_pad - B), (0, 0)))

    band1 = _build_band1(conv1_w).astype(jnp.bfloat16)
    b1v = jnp.pad(jnp.repeat(conv1_b, 12), (0, LN - 120)).reshape(1, LN)
    band2 = _build_band2(conv2_w).astype(jnp.bfloat16)
    b2v = jnp.pad(jnp.repeat(conv2_b, 4), (0, LN - 80)).reshape(1, LN)
    wf1 = jnp.transpose(fc1_w.reshape(50, 20, 4, 4), (2, 1, 3, 0)).reshape(4, 80, 50)
    wf1 = jnp.pad(wf1, ((0, 0), (0, LN - 80), (0, LN - 50)))
    bf1 = jnp.pad(fc1_b, (0, LN - 50)).reshape(1, LN)
    wf2 = jnp.pad(fc2_w.T, ((0, LN - 50), (0, LN - 10)))
    bf2 = jnp.pad(fc2_b, (0, LN - 10)).reshape(1, LN)

    logits = pl.pallas_call(
        _net_kernel,
        out_shape=jax.ShapeDtypeStruct((b_pad, LN), jnp.float32),
        grid=(n_tiles,),
        in_specs=[
            pl.BlockSpec((28, BT, 28), lambda i: (0, i, 0)),
            pl.BlockSpec((5, 28, 256), lambda i: (0, 0, 0)),
            pl.BlockSpec((1, LN), lambda i: (0, 0)),
            pl.BlockSpec((5, 128, 256), lambda i: (0, 0, 0)),
            pl.BlockSpec((1, LN), lambda i: (0, 0)),
            pl.BlockSpec((4, LN, LN), lambda i: (0, 0, 0)),
            pl.BlockSpec((1, LN), lambda i: (0, 0)),
            pl.BlockSpec((LN, LN), lambda i: (0, 0)),
            pl.BlockSpec((1, LN), lambda i: (0, 0)),
        ],
        out_specs=pl.BlockSpec((BT, LN), lambda i: (i, 0)),
        scratch_shapes=[pltpu.VMEM((12 * BT, LN), jnp.float32),
                        pltpu.VMEM((4 * BT, LN), jnp.float32)],
        compiler_params=pltpu.CompilerParams(
            dimension_semantics=("parallel",),
            vmem_limit_bytes=64 * 1024 * 1024,
        ),
    )(xt, band1, b1v, band2, b2v, wf1, bf1, wf2, bf2)

    return pl.pallas_call(
        _log_softmax_dim0_kernel,
        out_shape=jax.ShapeDtypeStruct((B, 10), jnp.float32),
        in_specs=[pl.BlockSpec(memory_space=pltpu.MemorySpace.VMEM)],
        out_specs=pl.BlockSpec(memory_space=pltpu.MemorySpace.VMEM),
    )(logits[:B])


# speedup vs baseline: 1.0243x; 1.0243x over previous
import numpy as np
import jax
import jax.numpy as jnp
from jax import lax
from jax.experimental import pallas as pl
from jax.experimental.pallas import tpu as pltpu

BT = 512
LN = 128


def _net_kernel(x_ref, band_ref, b1_ref, w2_ref, b2_ref,
                wf1_ref, bf1_ref, wf2_ref, bf2_ref, out_ref,
                h1_ref, h2_ref):
    hp = lax.Precision.DEFAULT
    f32 = jnp.float32
    X = x_ref[...].reshape(28 * BT, 28).astype(jnp.bfloat16)

    for r in range(12):
        acc = jnp.zeros((2 * BT, 256), f32)
        for ki in range(5):
            acc = acc + jnp.dot(X[(2 * r + ki) * BT:(2 * r + ki + 2) * BT],
                                band_ref[ki],
                                precision=hp, preferred_element_type=f32)
        v = acc.reshape(2, BT, 256)
        m = jnp.maximum(v[0], v[1])
        m = jnp.maximum(m[:, :LN], m[:, LN:])
        h1_ref[r * BT:(r + 1) * BT] = jnp.maximum(m + b1_ref[...], 0.0)

    for rr in range(4):
        acc = jnp.zeros((2 * BT, 256), f32)
        for ki in range(5):
            acc = acc + jnp.dot(
                h1_ref[(2 * rr + ki) * BT:(2 * rr + ki + 2) * BT], w2_ref[ki],
                precision=hp, preferred_element_type=f32)
        v = acc.reshape(2, BT, 256)
        m = jnp.maximum(v[0], v[1])
        m = jnp.maximum(m[:, :LN], m[:, LN:])
        h2_ref[rr * BT:(rr + 1) * BT] = jnp.maximum(m + b2_ref[...], 0.0)

    accf = jnp.zeros((BT, LN), f32)
    for ii in range(4):
        accf = accf + jnp.dot(h2_ref[ii * BT:(ii + 1) * BT], wf1_ref[ii],
                              precision=hp, preferred_element_type=f32)
    f1 = jnp.maximum(accf + bf1_ref[...], 0.0)

    out_ref[...] = jnp.dot(f1, wf2_ref[...], precision=hp,
                           preferred_element_type=f32) + bf2_ref[...]


def _log_softmax_dim0_kernel(x_ref, o_ref):
    x = x_ref[...]
    m = jnp.max(x, axis=0, keepdims=True)
    s = x - m
    lse = jnp.log(jnp.sum(jnp.exp(s), axis=0, keepdims=True))
    o_ref[...] = (s - lse)[:, :10]


def _onehot(w_size, m_size, parity):
    kj = np.arange(5)[:, None, None]
    w = np.arange(w_size)[None, :, None]
    m = np.arange(m_size)[None, None, :]
    return (w == 2 * m + parity + kj).astype(np.float32)


def _build_band1(w1):
    w1r = w1[:, 0]
    halves = []
    for parity in (0, 1):
        e = _onehot(28, 12, parity)
        h = jnp.einsum('cij,jwm->iwcm', w1r, e).reshape(5, 28, 120)
        halves.append(jnp.pad(h, ((0, 0), (0, 0), (0, LN - 120))))
    return jnp.concatenate(halves, axis=2)


def _build_band2(w2):
    halves = []
    for parity in (0, 1):
        e = _onehot(12, 4, parity)
        h = jnp.einsum('ocij,jwm->icwom', w2, e).reshape(5, 120, 80)
        halves.append(jnp.pad(h, ((0, 0), (0, 8), (0, LN - 80))))
    return jnp.concatenate(halves, axis=2)


def kernel(x, conv1_w, conv1_b, conv2_w, conv2_b, fc1_w, fc1_b, fc2_w, fc2_b):
    B = x.shape[0]
    n_tiles = (B + BT - 1) // BT
    b_pad = n_tiles * BT

    xt = jnp.transpose(x.reshape(B, 28, 28), (1, 0, 2))
    if b_pad != B:
        xt = jnp.pad(xt, ((0, 0), (0, b_pad - B), (0, 0)))

    band1 = _build_band1(conv1_w).astype(jnp.bfloat16)
    b1v = jnp.pad(jnp.repeat(conv1_b, 12), (0, LN - 120)).reshape(1, LN)
    band2 = _build_band2(conv2_w).astype(jnp.bfloat16)
    b2v = jnp.pad(jnp.repeat(conv2_b, 4), (0, LN - 80)).reshape(1, LN)
    wf1 = jnp.transpose(fc1_w.reshape(50, 20, 4, 4), (2, 1, 3, 0)).reshape(4, 80, 50)
    wf1 = jnp.pad(wf1, ((0, 0), (0, LN - 80), (0, LN - 50)))
    bf1 = jnp.pad(fc1_b, (0, LN - 50)).reshape(1, LN)
    wf2 = jnp.pad(fc2_w.T, ((0, LN - 50), (0, LN - 10)))
    bf2 = jnp.pad(fc2_b, (0, LN - 10)).reshape(1, LN)

    logits = pl.pallas_call(
        _net_kernel,
        out_shape=jax.ShapeDtypeStruct((b_pad, LN), jnp.float32),
        grid=(n_tiles,),
        in_specs=[
            pl.BlockSpec((28, BT, 28), lambda i: (0, i, 0)),
            pl.BlockSpec((5, 28, 256), lambda i: (0, 0, 0)),
            pl.BlockSpec((1, LN), lambda i: (0, 0)),
            pl.BlockSpec((5, 128, 256), lambda i: (0, 0, 0)),
            pl.BlockSpec((1, LN), lambda i: (0, 0)),
            pl.BlockSpec((4, LN, LN), lambda i: (0, 0, 0)),
            pl.BlockSpec((1, LN), lambda i: (0, 0)),
            pl.BlockSpec((LN, LN), lambda i: (0, 0)),
            pl.BlockSpec((1, LN), lambda i: (0, 0)),
        ],
        out_specs=pl.BlockSpec((BT, LN), lambda i: (i, 0)),
        scratch_shapes=[pltpu.VMEM((12 * BT, LN), jnp.float32),
                        pltpu.VMEM((4 * BT, LN), jnp.float32)],
        compiler_params=pltpu.CompilerParams(
            dimension_semantics=("parallel",),
            vmem_limit_bytes=64 * 1024 * 1024,
        ),
    )(xt, band1, b1v, band2, b2v, wf1, bf1, wf2, bf2)

    return pl.pallas_call(
        _log_softmax_dim0_kernel,
        out_shape=jax.ShapeDtypeStruct((B, 10), jnp.float32),
        in_specs=[pl.BlockSpec(memory_space=pltpu.MemorySpace.VMEM)],
        out_specs=pl.BlockSpec(memory_space=pltpu.MemorySpace.VMEM),
    )(logits[:B])
